# baseline (speedup 1.0000x reference)
import functools

import jax
import jax.numpy as jnp
from jax import lax
from jax.experimental import pallas as pl
from jax.experimental.pallas import tpu as pltpu
from jax.experimental.pallas import tpu_sc as plsc


def _proj_body(table_ref, w_ref, b_ref, out_ref):
    out_ref[...] = (
        jnp.dot(table_ref[...], w_ref[...], preferred_element_type=jnp.float32)
        + b_ref[...]
    )


@functools.lru_cache(maxsize=None)
def _make_project(V, D, blk):
    grid = V // blk
    return pl.pallas_call(
        _proj_body,
        grid=(grid,),
        in_specs=[
            pl.BlockSpec((blk, D), lambda i: (i, 0)),
            pl.BlockSpec((D, D), lambda i: (0, 0)),
            pl.BlockSpec((1, D), lambda i: (0, 0)),
        ],
        out_specs=pl.BlockSpec((blk, D), lambda i: (i, 0)),
        out_shape=jax.ShapeDtypeStruct((V, D), jnp.float32),
    )


@functools.lru_cache(maxsize=None)
def _make_gather(V, D, B, L):
    info = plsc.get_sparse_core_info()
    NC, NS, LN = info.num_cores, info.num_subcores, info.num_lanes
    NW = NC * NS
    BW = B // NW
    assert BW == 128 and D == 64 and LN == 16
    NBUF = 2
    mesh = plsc.VectorSubcoreMesh(core_axis_name="c", subcore_axis_name="s")

    @functools.partial(
        pl.kernel,
        mesh=mesh,
        compiler_params=pltpu.CompilerParams(
            use_tc_tiling_on_sc=False, needs_layout_passes=False),
        out_type=jax.ShapeDtypeStruct((L, D // 8, NW, 8 * BW), jnp.float32),
        scratch_types=[
            pltpu.VMEM((BW, L), jnp.int32),
            pltpu.VMEM((L * BW,), jnp.int32),
            pltpu.VMEM((BW, D), jnp.float32),
            pltpu.VMEM((BW, D), jnp.float32),
            pltpu.VMEM((D * BW,), jnp.float32),
            pltpu.VMEM((D * BW,), jnp.float32),
            pltpu.SemaphoreType.DMA,
            pltpu.SemaphoreType.DMA,
            pltpu.SemaphoreType.DMA,
        ],
    )
    def gather_kernel(table_hbm, idx_hbm, out_hbm, idx_v, idxT_v,
                      g0, g1, t0, t1, gsem, w0, w1):
        gbuf = (g0, g1)
        tbuf = (t0, t1)
        wsem = (w0, w1)
        wid = lax.axis_index("s") * NC + lax.axis_index("c")
        base = wid * BW
        lane = lax.iota(jnp.int32, 16)

        def fire_gather(l, b):
            pltpu.async_copy(
                table_hbm.at[idxT_v.at[pl.ds(l * BW, BW)]], gbuf[b], gsem)

        def wait_gather(b):
            pltpu.make_async_copy(
                table_hbm.at[pl.ds(0, BW)], gbuf[b], gsem).wait()

        def fire_wb(l, b):
            for tr in range(D // 8):
                pltpu.async_copy(
                    tbuf[b].at[pl.ds(tr * 8 * BW, 8 * BW)],
                    out_hbm.at[l, tr, wid], wsem[b])

        def wait_wb(b):
            for tr in range(D // 8):
                pltpu.make_async_copy(
                    tbuf[b].at[pl.ds(tr * 8 * BW, 8 * BW)],
                    out_hbm.at[0, tr, 0], wsem[b]).wait()

        pltpu.sync_copy(idx_hbm.at[pl.ds(base, BW), :], idx_v)

        offs = list(range(0, L - 15, 16)) + [L - 16]

        def idx_t_body(j, carry):
            for off in offs:
                v = idx_v[j, pl.ds(off, 16)]
                plsc.store_scatter(idxT_v, [(lane + off) * BW + j], v)
            return carry

        lax.fori_loop(0, BW, idx_t_body, 0)

        for b in range(NBUF):
            fire_gather(b, b)

        def body(i, carry):
            for b in range(NBUF):
                l = i * NBUF + b
                wait_gather(b)

                @pl.when(l >= NBUF)
                def _():
                    wait_wb(b)

                def tr_body(jg, carry2):
                    j0 = jg * 8
                    for dj in range(8):
                        j = j0 + dj
                        for k in range(D // 16):
                            v = gbuf[b][j, pl.ds(16 * k, 16)]
                            plsc.store_scatter(
                                tbuf[b], [(lane + 16 * k) * BW + j], v)
                    return carry2

                lax.fori_loop(0, BW // 8, tr_body, 0)
                fire_wb(l, b)
                nxt = l + NBUF

                @pl.when(nxt < L)
                def _():
                    fire_gather(nxt, b)

            return carry

        lax.fori_loop(0, L // NBUF, body, 0)
        for b in range(NBUF):
            wait_wb(b)

    return gather_kernel


def kernel(x, table, W, b):
    B, L = x.shape
    V, D = table.shape
    proj = _make_project(V, D, 2000)(table, W, b.reshape(1, D))
    out4 = _make_gather(V, D, B, L)(proj, x.astype(jnp.int32))
    out5 = out4.reshape(L, D // 8, B // 128, 8, 128)
    return out5.transpose(2, 4, 0, 1, 3).reshape(B, L, D)


# speedup vs baseline: 2.0803x; 2.0803x over previous
import functools

import jax
import jax.numpy as jnp
from jax import lax
from jax.experimental import pallas as pl
from jax.experimental.pallas import tpu as pltpu
from jax.experimental.pallas import tpu_sc as plsc


def _proj_body(table_ref, w_ref, b_ref, out_ref):
    out_ref[...] = (
        jnp.dot(table_ref[...], w_ref[...], preferred_element_type=jnp.float32)
        + b_ref[...]
    )


@functools.lru_cache(maxsize=None)
def _make_project(V, D, blk):
    grid = V // blk
    return pl.pallas_call(
        _proj_body,
        grid=(grid,),
        in_specs=[
            pl.BlockSpec((blk, D), lambda i: (i, 0)),
            pl.BlockSpec((D, D), lambda i: (0, 0)),
            pl.BlockSpec((1, D), lambda i: (0, 0)),
        ],
        out_specs=pl.BlockSpec((blk, D), lambda i: (i, 0)),
        out_shape=jax.ShapeDtypeStruct((V, D), jnp.float32),
    )


@functools.lru_cache(maxsize=None)
def _make_gather(V, D, B, L):
    info = plsc.get_sparse_core_info()
    NC, NS, LN = info.num_cores, info.num_subcores, info.num_lanes
    NW = NC * NS
    BW = B // NW
    assert BW == 128 and D == 64 and LN == 16
    NBUF = 2
    mesh = plsc.VectorSubcoreMesh(core_axis_name="c", subcore_axis_name="s")

    @functools.partial(
        pl.kernel,
        mesh=mesh,
        compiler_params=pltpu.CompilerParams(
            use_tc_tiling_on_sc=False, needs_layout_passes=False),
        out_type=jax.ShapeDtypeStruct((L, D // 8, NW, 8 * BW), jnp.float32),
        scratch_types=[
            pltpu.VMEM((BW, L), jnp.int32),
            pltpu.VMEM((L * BW,), jnp.int32),
            pltpu.VMEM((BW, D), jnp.float32),
            pltpu.VMEM((BW, D), jnp.float32),
            pltpu.VMEM((D * BW,), jnp.float32),
            pltpu.VMEM((D * BW,), jnp.float32),
            pltpu.SemaphoreType.DMA,
            pltpu.SemaphoreType.DMA,
            pltpu.SemaphoreType.DMA,
        ],
    )
    def gather_kernel(table_hbm, idx_hbm, out_hbm, idx_v, idxT_v,
                      g0, g1, t0, t1, gsem, w0, w1):
        gbuf = (g0, g1)
        tbuf = (t0, t1)
        wsem = (w0, w1)
        wid = lax.axis_index("s") * NC + lax.axis_index("c")
        base = wid * BW
        lane = lax.iota(jnp.int32, 16)

        def fire_gather(l, b):
            pltpu.async_copy(
                table_hbm.at[idxT_v.at[pl.ds(l * BW, BW)]], gbuf[b], gsem)

        def wait_gather(b):
            pltpu.make_async_copy(
                table_hbm.at[pl.ds(0, BW)], gbuf[b], gsem).wait()

        def fire_wb(l, b):
            for tr in range(D // 8):
                pltpu.async_copy(
                    tbuf[b].at[pl.ds(tr * 8 * BW, 8 * BW)],
                    out_hbm.at[l, tr, wid], wsem[b])

        def wait_wb(b):
            for tr in range(D // 8):
                pltpu.make_async_copy(
                    tbuf[b].at[pl.ds(tr * 8 * BW, 8 * BW)],
                    out_hbm.at[0, tr, 0], wsem[b]).wait()

        pltpu.sync_copy(idx_hbm.at[pl.ds(base, BW), :], idx_v)

        offs = list(range(0, L - 15, 16)) + [L - 16]

        def idx_t_body(j, carry):
            for off in offs:
                v = idx_v[j, pl.ds(off, 16)]
                plsc.store_scatter(idxT_v, [(lane + off) * BW + j], v)
            return carry

        lax.fori_loop(0, BW, idx_t_body, 0)

        for b in range(NBUF):
            fire_gather(b, b)

        def body(i, carry):
            for b in range(NBUF):
                l = i * NBUF + b
                wait_gather(b)

                @pl.when(l >= NBUF)
                def _():
                    wait_wb(b)

                def tr_body(m, carry2):
                    perm = lane + m
                    perm = jnp.where(perm < 16, perm, perm - 16)
                    for j0 in range(0, BW, 16):
                        rowv = perm + j0
                        for k in range(D // 16):
                            v = plsc.load_gather(
                                gbuf[b], [rowv, lane + 16 * k])
                            plsc.store_scatter(
                                tbuf[b], [(lane + 16 * k) * BW + rowv], v)
                    return carry2

                lax.fori_loop(0, 16, tr_body, 0)
                fire_wb(l, b)
                nxt = l + NBUF

                @pl.when(nxt < L)
                def _():
                    fire_gather(nxt, b)

            return carry

        lax.fori_loop(0, L // NBUF, body, 0)
        for b in range(NBUF):
            wait_wb(b)

    return gather_kernel


def kernel(x, table, W, b):
    B, L = x.shape
    V, D = table.shape
    proj = _make_project(V, D, 2000)(table, W, b.reshape(1, D))
    out4 = _make_gather(V, D, B, L)(proj, x.astype(jnp.int32))
    out5 = out4.reshape(L, D // 8, B // 128, 8, 128)
    return out5.transpose(2, 4, 0, 1, 3).reshape(B, L, D)


# speedup vs baseline: 2.3528x; 1.1310x over previous
import functools

import jax
import jax.numpy as jnp
from jax import lax
from jax.experimental import pallas as pl
from jax.experimental.pallas import tpu as pltpu
from jax.experimental.pallas import tpu_sc as plsc


def _proj_body(tableT_ref, w_ref, b_ref, out_ref):
    out_ref[...] = lax.dot_general(
        tableT_ref[...], w_ref[...], (((0,), (0,)), ((), ())),
        preferred_element_type=jnp.float32,
    ) + b_ref[...]


@functools.lru_cache(maxsize=None)
def _make_project(V, D, blk):
    grid = (V + blk - 1) // blk
    return pl.pallas_call(
        _proj_body,
        grid=(grid,),
        in_specs=[
            pl.BlockSpec((D, blk), lambda i: (0, i)),
            pl.BlockSpec((D, D), lambda i: (0, 0)),
            pl.BlockSpec((1, D), lambda i: (0, 0)),
        ],
        out_specs=pl.BlockSpec((blk, D), lambda i: (i, 0)),
        out_shape=jax.ShapeDtypeStruct((V, D), jnp.float32),
    )


@functools.lru_cache(maxsize=None)
def _make_gather(V, D, B, L):
    info = plsc.get_sparse_core_info()
    NC, NS, LN = info.num_cores, info.num_subcores, info.num_lanes
    NW = NC * NS
    BW = B // NW
    assert BW == 128 and D == 64 and LN == 16
    NBUF = 2
    mesh = plsc.VectorSubcoreMesh(core_axis_name="c", subcore_axis_name="s")

    @functools.partial(
        pl.kernel,
        mesh=mesh,
        compiler_params=pltpu.CompilerParams(
            use_tc_tiling_on_sc=False, needs_layout_passes=False),
        out_type=jax.ShapeDtypeStruct((L, D // 8, NW, 8 * BW), jnp.float32),
        scratch_types=[
            pltpu.VMEM((BW, L), jnp.int32),
            pltpu.VMEM((L * BW,), jnp.int32),
            pltpu.VMEM((BW, D), jnp.float32),
            pltpu.VMEM((BW, D), jnp.float32),
            pltpu.VMEM((D * BW,), jnp.float32),
            pltpu.VMEM((D * BW,), jnp.float32),
            pltpu.SemaphoreType.DMA,
            pltpu.SemaphoreType.DMA,
            pltpu.SemaphoreType.DMA,
        ],
    )
    def gather_kernel(table_hbm, idx_hbm, out_hbm, idx_v, idxT_v,
                      g0, g1, t0, t1, gsem, w0, w1):
        gbuf = (g0, g1)
        tbuf = (t0, t1)
        wsem = (w0, w1)
        wid = lax.axis_index("s") * NC + lax.axis_index("c")
        base = wid * BW
        lane = lax.iota(jnp.int32, 16)

        def fire_gather(l, b):
            pltpu.async_copy(
                table_hbm.at[idxT_v.at[pl.ds(l * BW, BW)]], gbuf[b], gsem)

        def wait_gather(b):
            pltpu.make_async_copy(
                table_hbm.at[pl.ds(0, BW)], gbuf[b], gsem).wait()

        def fire_wb(l, b):
            for tr in range(D // 8):
                pltpu.async_copy(
                    tbuf[b].at[pl.ds(tr * 8 * BW, 8 * BW)],
                    out_hbm.at[l, tr, wid], wsem[b])

        def wait_wb(b):
            for tr in range(D // 8):
                pltpu.make_async_copy(
                    tbuf[b].at[pl.ds(tr * 8 * BW, 8 * BW)],
                    out_hbm.at[0, tr, 0], wsem[b]).wait()

        pltpu.sync_copy(idx_hbm.at[pl.ds(base, BW), :], idx_v)

        offs = list(range(0, L - 15, 16)) + [L - 16]

        def idx_t_body(j, carry):
            for off in offs:
                v = idx_v[j, pl.ds(off, 16)]
                plsc.store_scatter(idxT_v, [(lane + off) * BW + j], v)
            return carry

        lax.fori_loop(0, BW, idx_t_body, 0)

        for b in range(NBUF):
            fire_gather(b, b)

        def body(i, carry):
            for b in range(NBUF):
                l = i * NBUF + b
                wait_gather(b)

                @pl.when(l >= NBUF)
                def _():
                    wait_wb(b)

                def tr_body(m, carry2):
                    perm = lane + m
                    perm = jnp.where(perm < 16, perm, perm - 16)
                    for j0 in range(0, BW, 16):
                        rowv = perm + j0
                        for k in range(D // 16):
                            v = plsc.load_gather(
                                gbuf[b], [rowv, lane + 16 * k])
                            plsc.store_scatter(
                                tbuf[b], [(lane + 16 * k) * BW + rowv], v)
                    return carry2

                lax.fori_loop(0, 16, tr_body, 0)
                fire_wb(l, b)
                nxt = l + NBUF

                @pl.when(nxt < L)
                def _():
                    fire_gather(nxt, b)

            return carry

        lax.fori_loop(0, L // NBUF, body, 0)
        for b in range(NBUF):
            wait_wb(b)

    return gather_kernel


def kernel(x, table, W, b):
    B, L = x.shape
    V, D = table.shape
    proj = _make_project(V, D, 12800)(table.T, W, b.reshape(1, D))
    out4 = _make_gather(V, D, B, L)(proj, x.astype(jnp.int32))
    out5 = out4.reshape(L, D // 8, B // 128, 8, 128)
    return out5.transpose(2, 4, 0, 1, 3).reshape(B, L, D)


# speedup vs baseline: 3.6111x; 1.5348x over previous
import functools

import jax
import jax.numpy as jnp
from jax import lax
from jax.experimental import pallas as pl
from jax.experimental.pallas import tpu as pltpu
from jax.experimental.pallas import tpu_sc as plsc


def _proj_body(tableT_ref, w_ref, b_ref, out_ref):
    out_ref[...] = lax.dot_general(
        tableT_ref[...], w_ref[...], (((0,), (0,)), ((), ())),
        preferred_element_type=jnp.float32,
    ) + b_ref[...]


@functools.lru_cache(maxsize=None)
def _make_project(V, D, blk):
    grid = (V + blk - 1) // blk
    return pl.pallas_call(
        _proj_body,
        grid=(grid,),
        in_specs=[
            pl.BlockSpec((D, blk), lambda i: (0, i)),
            pl.BlockSpec((D, D), lambda i: (0, 0)),
            pl.BlockSpec((1, D), lambda i: (0, 0)),
        ],
        out_specs=pl.BlockSpec((blk, D), lambda i: (i, 0)),
        out_shape=jax.ShapeDtypeStruct((V, D), jnp.float32),
    )


@functools.lru_cache(maxsize=None)
def _make_gather(V, D, B, L):
    info = plsc.get_sparse_core_info()
    NC, NS, LN = info.num_cores, info.num_subcores, info.num_lanes
    NW = NC * NS
    BW = B // NW
    assert BW == 128 and D == 64 and LN == 16
    NBUF = 2
    mesh = plsc.VectorSubcoreMesh(core_axis_name="c", subcore_axis_name="s")

    @functools.partial(
        pl.kernel,
        mesh=mesh,
        compiler_params=pltpu.CompilerParams(
            use_tc_tiling_on_sc=False, needs_layout_passes=False),
        out_type=jax.ShapeDtypeStruct((L, D // 8, NW, 8 * BW), jnp.float32),
        scratch_types=[
            pltpu.VMEM((BW, L), jnp.int32),
            pltpu.VMEM((L * BW,), jnp.int32),
            pltpu.VMEM((BW, D), jnp.float32),
            pltpu.VMEM((BW, D), jnp.float32),
            pltpu.VMEM((D * BW,), jnp.float32),
            pltpu.VMEM((D * BW,), jnp.float32),
            pltpu.SemaphoreType.DMA,
            pltpu.SemaphoreType.DMA,
            pltpu.SemaphoreType.DMA,
        ],
    )
    def gather_kernel(table_hbm, idx_hbm, out_hbm, idx_v, idxT_v,
                      g0, g1, t0, t1, gsem, w0, w1):
        gbuf = (g0, g1)
        tbuf = (t0, t1)
        wsem = (w0, w1)
        wid = lax.axis_index("s") * NC + lax.axis_index("c")
        base = wid * BW
        lane = lax.iota(jnp.int32, 16)

        def fire_gather(l, b):
            pltpu.async_copy(
                table_hbm.at[idxT_v.at[pl.ds(l * BW, BW)]], gbuf[b], gsem)

        def wait_gather(b):
            pltpu.make_async_copy(
                table_hbm.at[pl.ds(0, BW)], gbuf[b], gsem).wait()

        def fire_wb(l, b):
            for tr in range(D // 8):
                pltpu.async_copy(
                    tbuf[b].at[pl.ds(tr * 8 * BW, 8 * BW)],
                    out_hbm.at[l, tr, wid], wsem[b])

        def wait_wb(b):
            for tr in range(D // 8):
                pltpu.make_async_copy(
                    tbuf[b].at[pl.ds(tr * 8 * BW, 8 * BW)],
                    out_hbm.at[0, tr, 0], wsem[b]).wait()

        pltpu.sync_copy(idx_hbm.at[pl.ds(base, BW), :], idx_v)

        offs = list(range(0, L - 15, 16)) + [L - 16]

        def idx_t_body(j, carry):
            for off in offs:
                v = idx_v[j, pl.ds(off, 16)]
                plsc.store_scatter(idxT_v, [(lane + off) * BW + j], v)
            return carry

        lax.fori_loop(0, BW, idx_t_body, 0)

        for b in range(NBUF):
            fire_gather(b, b)

        def body(i, carry):
            for b in range(NBUF):
                l = i * NBUF + b
                wait_gather(b)

                @pl.when(l >= NBUF)
                def _():
                    wait_wb(b)

                def tr_body(m, carry2):
                    perm = lane + m
                    perm = jnp.where(perm < 16, perm, perm - 16)
                    for jp in range(0, BW, 32):
                        batch = []
                        for j0 in (jp, jp + 16):
                            rowv = perm + j0
                            for k in range(D // 16):
                                v = plsc.load_gather(
                                    gbuf[b], [rowv, lane + 16 * k])
                                batch.append(
                                    ((lane + 16 * k) * BW + rowv, v))
                        for sidx, v in batch:
                            plsc.store_scatter(tbuf[b], [sidx], v)
                    return carry2

                lax.fori_loop(0, 16, tr_body, 0)
                fire_wb(l, b)
                nxt = l + NBUF

                @pl.when(nxt < L)
                def _():
                    fire_gather(nxt, b)

            return carry

        lax.fori_loop(0, L // NBUF, body, 0)
        for b in range(NBUF):
            wait_wb(b)

    return gather_kernel


def kernel(x, table, W, b):
    B, L = x.shape
    V, D = table.shape
    proj = _make_project(V, D, 12800)(table.T, W, b.reshape(1, D))
    out4 = _make_gather(V, D, B, L)(proj, x.astype(jnp.int32))
    out5 = out4.reshape(L, D // 8, B // 128, 8, 128)
    return out5.transpose(2, 4, 0, 1, 3).reshape(B, L, D)


# speedup vs baseline: 4.2539x; 1.1780x over previous
import functools

import jax
import jax.numpy as jnp
from jax import lax
from jax.experimental import pallas as pl
from jax.experimental.pallas import tpu as pltpu
from jax.experimental.pallas import tpu_sc as plsc


def _proj_body(tableT_ref, w_ref, b_ref, out_ref):
    out_ref[...] = lax.dot_general(
        tableT_ref[...], w_ref[...], (((0,), (0,)), ((), ())),
        preferred_element_type=jnp.float32,
    ) + b_ref[...]


@functools.lru_cache(maxsize=None)
def _make_project(V, D, blk):
    grid = (V + blk - 1) // blk
    return pl.pallas_call(
        _proj_body,
        grid=(grid,),
        in_specs=[
            pl.BlockSpec((D, blk), lambda i: (0, i)),
            pl.BlockSpec((D, D), lambda i: (0, 0)),
            pl.BlockSpec((1, D), lambda i: (0, 0)),
        ],
        out_specs=pl.BlockSpec((blk, D), lambda i: (i, 0)),
        out_shape=jax.ShapeDtypeStruct((V, D), jnp.float32),
    )


@functools.lru_cache(maxsize=None)
def _make_gather(V, D, B, L):
    info = plsc.get_sparse_core_info()
    NC, NS, LN = info.num_cores, info.num_subcores, info.num_lanes
    NW = NC * NS
    BW = B // NW
    assert BW == 128 and D == 64 and LN == 16
    NBUF = 4
    mesh = plsc.VectorSubcoreMesh(core_axis_name="c", subcore_axis_name="s")

    @functools.partial(
        pl.kernel,
        mesh=mesh,
        compiler_params=pltpu.CompilerParams(
            use_tc_tiling_on_sc=False, needs_layout_passes=False),
        out_type=jax.ShapeDtypeStruct((L, D // 8, NW, 8 * BW), jnp.float32),
        scratch_types=[
            pltpu.VMEM((BW, L), jnp.int32),
            pltpu.VMEM((L * BW,), jnp.int32),
            pltpu.VMEM((BW, D), jnp.float32),
            pltpu.VMEM((BW, D), jnp.float32),
            pltpu.VMEM((BW, D), jnp.float32),
            pltpu.VMEM((BW, D), jnp.float32),
            pltpu.VMEM((D * BW,), jnp.float32),
            pltpu.VMEM((D * BW,), jnp.float32),
            pltpu.VMEM((D * BW,), jnp.float32),
            pltpu.VMEM((D * BW,), jnp.float32),
            pltpu.SemaphoreType.DMA,
            pltpu.SemaphoreType.DMA,
            pltpu.SemaphoreType.DMA,
            pltpu.SemaphoreType.DMA,
            pltpu.SemaphoreType.DMA,
        ],
    )
    def gather_kernel(table_hbm, idx_hbm, out_hbm, idx_v, idxT_v,
                      g0, g1, g2, g3, t0, t1, t2, t3, gsem, w0, w1, w2, w3):
        gbuf = (g0, g1, g2, g3)
        tbuf = (t0, t1, t2, t3)
        wsem = (w0, w1, w2, w3)
        wid = lax.axis_index("s") * NC + lax.axis_index("c")
        base = wid * BW
        lane = lax.iota(jnp.int32, 16)

        def fire_gather(l, b):
            pltpu.async_copy(
                table_hbm.at[idxT_v.at[pl.ds(l * BW, BW)]], gbuf[b], gsem)

        def wait_gather(b):
            pltpu.make_async_copy(
                table_hbm.at[pl.ds(0, BW)], gbuf[b], gsem).wait()

        def fire_wb(l, b):
            for tr in range(D // 8):
                pltpu.async_copy(
                    tbuf[b].at[pl.ds(tr * 8 * BW, 8 * BW)],
                    out_hbm.at[l, tr, wid], wsem[b])

        def wait_wb(b):
            for tr in range(D // 8):
                pltpu.make_async_copy(
                    tbuf[b].at[pl.ds(tr * 8 * BW, 8 * BW)],
                    out_hbm.at[0, tr, 0], wsem[b]).wait()

        pltpu.sync_copy(idx_hbm.at[pl.ds(base, BW), :], idx_v)

        offs = list(range(0, L - 15, 16)) + [L - 16]

        def idx_t_body(j, carry):
            for off in offs:
                v = idx_v[j, pl.ds(off, 16)]
                plsc.store_scatter(idxT_v, [(lane + off) * BW + j], v)
            return carry

        lax.fori_loop(0, BW, idx_t_body, 0)

        for b in range(NBUF):
            fire_gather(b, b)

        def body(i, carry):
            for b in range(NBUF):
                l = i * NBUF + b
                wait_gather(b)

                @pl.when(l >= NBUF)
                def _():
                    wait_wb(b)

                def tr_body(m, carry2):
                    perm = lane + m
                    perm = jnp.where(perm < 16, perm, perm - 16)
                    for jp in range(0, BW, 32):
                        batch = []
                        for j0 in (jp, jp + 16):
                            rowv = perm + j0
                            for k in range(D // 16):
                                v = plsc.load_gather(
                                    gbuf[b], [rowv, lane + 16 * k])
                                batch.append(
                                    ((lane + 16 * k) * BW + rowv, v))
                        for sidx, v in batch:
                            plsc.store_scatter(tbuf[b], [sidx], v)
                    return carry2

                lax.fori_loop(0, 16, tr_body, 0)
                fire_wb(l, b)
                nxt = l + NBUF

                @pl.when(nxt < L)
                def _():
                    fire_gather(nxt, b)

            return carry

        lax.fori_loop(0, L // NBUF, body, 0)
        for b in range(NBUF):
            wait_wb(b)

    return gather_kernel


def kernel(x, table, W, b):
    B, L = x.shape
    V, D = table.shape
    proj = _make_project(V, D, 12800)(table.T, W, b.reshape(1, D))
    out4 = _make_gather(V, D, B, L)(proj, x.astype(jnp.int32))
    out5 = out4.reshape(L, D // 8, B // 128, 8, 128)
    return out5.transpose(2, 4, 0, 1, 3).reshape(B, L, D)


# speedup vs baseline: 4.8641x; 1.1434x over previous
import functools

import jax
import jax.numpy as jnp
from jax import lax
from jax.experimental import pallas as pl
from jax.experimental.pallas import tpu as pltpu
from jax.experimental.pallas import tpu_sc as plsc


def _proj_body(tableT_ref, w_ref, b_ref, out_ref):
    out_ref[:, pl.ds(0, w_ref.shape[0])] = lax.dot_general(
        tableT_ref[...], w_ref[...], (((0,), (0,)), ((), ())),
        preferred_element_type=jnp.float32,
    ) + b_ref[...]


@functools.lru_cache(maxsize=None)
def _make_project(V, D, blk):
    grid = (V + blk - 1) // blk
    return pl.pallas_call(
        _proj_body,
        grid=(grid,),
        in_specs=[
            pl.BlockSpec((D, blk), lambda i: (0, i)),
            pl.BlockSpec((D, D), lambda i: (0, 0)),
            pl.BlockSpec((1, D), lambda i: (0, 0)),
        ],
        out_specs=pl.BlockSpec((blk, 2 * D), lambda i: (i, 0)),
        out_shape=jax.ShapeDtypeStruct((V, 2 * D), jnp.float32),
    )


@functools.lru_cache(maxsize=None)
def _make_gather(V, D, B, L):
    info = plsc.get_sparse_core_info()
    NC, NS, LN = info.num_cores, info.num_subcores, info.num_lanes
    NW = NC * NS
    BW = B // NW
    assert BW == 128 and D == 64 and LN == 16
    NBUF = 4
    mesh = plsc.VectorSubcoreMesh(core_axis_name="c", subcore_axis_name="s")

    @functools.partial(
        pl.kernel,
        mesh=mesh,
        compiler_params=pltpu.CompilerParams(
            use_tc_tiling_on_sc=False, needs_layout_passes=False),
        out_type=jax.ShapeDtypeStruct((L, D // 8, NW, 8 * BW), jnp.float32),
        scratch_types=[
            pltpu.VMEM((BW, L), jnp.int32),
            pltpu.VMEM((L * BW,), jnp.int32),
            pltpu.VMEM((BW, D), jnp.float32),
            pltpu.VMEM((BW, D), jnp.float32),
            pltpu.VMEM((BW, D), jnp.float32),
            pltpu.VMEM((BW, D), jnp.float32),
            pltpu.VMEM((D * BW,), jnp.float32),
            pltpu.VMEM((D * BW,), jnp.float32),
            pltpu.VMEM((D * BW,), jnp.float32),
            pltpu.VMEM((D * BW,), jnp.float32),
            pltpu.SemaphoreType.DMA,
            pltpu.SemaphoreType.DMA,
            pltpu.SemaphoreType.DMA,
            pltpu.SemaphoreType.DMA,
            pltpu.SemaphoreType.DMA,
        ],
    )
    def gather_kernel(table_hbm, idx_hbm, out_hbm, idx_v, idxT_v,
                      g0, g1, g2, g3, t0, t1, t2, t3, gsem, w0, w1, w2, w3):
        gbuf = (g0, g1, g2, g3)
        tbuf = (t0, t1, t2, t3)
        wsem = (w0, w1, w2, w3)
        wid = lax.axis_index("s") * NC + lax.axis_index("c")
        base = wid * BW
        lane = lax.iota(jnp.int32, 16)

        def fire_gather(l, b):
            pltpu.async_copy(
                table_hbm.at[idxT_v.at[pl.ds(l * BW, BW)]], gbuf[b], gsem)

        def wait_gather(b):
            pltpu.make_async_copy(
                table_hbm.at[pl.ds(0, BW)], gbuf[b], gsem).wait()

        def fire_wb(l, b):
            for tr in range(D // 8):
                pltpu.async_copy(
                    tbuf[b].at[pl.ds(tr * 8 * BW, 8 * BW)],
                    out_hbm.at[l, tr, wid], wsem[b])

        def wait_wb(b):
            for tr in range(D // 8):
                pltpu.make_async_copy(
                    tbuf[b].at[pl.ds(tr * 8 * BW, 8 * BW)],
                    out_hbm.at[0, tr, 0], wsem[b]).wait()

        pltpu.sync_copy(idx_hbm.at[pl.ds(base, BW), :], idx_v)

        offs = list(range(0, L - 15, 16)) + [L - 16]

        def idx_t_body(j, carry):
            for off in offs:
                v = idx_v[j, pl.ds(off, 16)]
                plsc.store_scatter(idxT_v, [(lane + off) * BW + j], v + v)
            return carry

        lax.fori_loop(0, BW, idx_t_body, 0)

        for b in range(NBUF):
            fire_gather(b, b)

        def body(i, carry):
            for b in range(NBUF):
                l = i * NBUF + b
                wait_gather(b)

                @pl.when(l >= NBUF)
                def _():
                    wait_wb(b)

                def tr_body(m, carry2):
                    perm = lane + m
                    perm = jnp.where(perm < 16, perm, perm - 16)
                    for jp in range(0, BW, 32):
                        batch = []
                        for j0 in (jp, jp + 16):
                            rowv = perm + j0
                            for k in range(D // 16):
                                v = plsc.load_gather(
                                    gbuf[b], [rowv, lane + 16 * k])
                                batch.append(
                                    ((lane + 16 * k) * BW + rowv, v))
                        for sidx, v in batch:
                            plsc.store_scatter(tbuf[b], [sidx], v)
                    return carry2

                lax.fori_loop(0, 16, tr_body, 0)
                fire_wb(l, b)
                nxt = l + NBUF

                @pl.when(nxt < L)
                def _():
                    fire_gather(nxt, b)

            return carry

        lax.fori_loop(0, L // NBUF, body, 0)
        for b in range(NBUF):
            wait_wb(b)

    return gather_kernel


def kernel(x, table, W, b):
    B, L = x.shape
    V, D = table.shape
    proj = _make_project(V, D, 12800)(table.T, W, b.reshape(1, D))
    out4 = _make_gather(V, D, B, L)(proj.reshape(2 * V, D), x.astype(jnp.int32))
    out5 = out4.reshape(L, D // 8, B // 128, 8, 128)
    return out5.transpose(2, 4, 0, 1, 3).reshape(B, L, D)


# speedup vs baseline: 4.8803x; 1.0033x over previous
import functools

import jax
import jax.numpy as jnp
from jax import lax
from jax.experimental import pallas as pl
from jax.experimental.pallas import tpu as pltpu
from jax.experimental.pallas import tpu_sc as plsc


def _proj_body(tableT_ref, w_ref, b_ref, out_ref):
    out_ref[:, pl.ds(0, w_ref.shape[0])] = lax.dot_general(
        tableT_ref[...], w_ref[...], (((0,), (0,)), ((), ())),
        preferred_element_type=jnp.float32,
    ) + b_ref[...]


@functools.lru_cache(maxsize=None)
def _make_project(V, D, blk):
    grid = (V + blk - 1) // blk
    return pl.pallas_call(
        _proj_body,
        grid=(grid,),
        in_specs=[
            pl.BlockSpec((D, blk), lambda i: (0, i)),
            pl.BlockSpec((D, D), lambda i: (0, 0)),
            pl.BlockSpec((1, D), lambda i: (0, 0)),
        ],
        out_specs=pl.BlockSpec((blk, 2 * D), lambda i: (i, 0)),
        out_shape=jax.ShapeDtypeStruct((V, 2 * D), jnp.float32),
    )


@functools.lru_cache(maxsize=None)
def _make_gather(V, D, B, L):
    info = plsc.get_sparse_core_info()
    NC, NS, LN = info.num_cores, info.num_subcores, info.num_lanes
    NW = NC * NS
    BW = B // NW
    assert BW == 128 and D == 64 and LN == 16
    NBUF = 4
    mesh = plsc.VectorSubcoreMesh(core_axis_name="c", subcore_axis_name="s")

    @functools.partial(
        pl.kernel,
        mesh=mesh,
        compiler_params=pltpu.CompilerParams(
            use_tc_tiling_on_sc=False, needs_layout_passes=False),
        out_type=jax.ShapeDtypeStruct((L, D // 8, NW, 8 * BW), jnp.float32),
        scratch_types=[
            pltpu.VMEM((BW, L), jnp.int32),
            pltpu.VMEM((L * BW,), jnp.int32),
            pltpu.VMEM((BW, D), jnp.float32),
            pltpu.VMEM((BW, D), jnp.float32),
            pltpu.VMEM((BW, D), jnp.float32),
            pltpu.VMEM((BW, D), jnp.float32),
            pltpu.VMEM((D * BW,), jnp.float32),
            pltpu.VMEM((D * BW,), jnp.float32),
            pltpu.VMEM((D * BW,), jnp.float32),
            pltpu.VMEM((D * BW,), jnp.float32),
            pltpu.SemaphoreType.DMA,
            pltpu.SemaphoreType.DMA,
            pltpu.SemaphoreType.DMA,
            pltpu.SemaphoreType.DMA,
            pltpu.SemaphoreType.DMA,
        ],
    )
    def gather_kernel(table_hbm, idx_hbm, out_hbm, idx_v, idxT_v,
                      g0, g1, g2, g3, t0, t1, t2, t3, gsem, w0, w1, w2, w3):
        gbuf = (g0, g1, g2, g3)
        tbuf = (t0, t1, t2, t3)
        wsem = (w0, w1, w2, w3)
        wid = lax.axis_index("s") * NC + lax.axis_index("c")
        base = wid * BW
        lane = lax.iota(jnp.int32, 16)

        def fire_gather(l, b):
            pltpu.async_copy(
                table_hbm.at[idxT_v.at[pl.ds(l * BW, BW)]], gbuf[b], gsem)

        def wait_gather(b):
            pltpu.make_async_copy(
                table_hbm.at[pl.ds(0, BW)], gbuf[b], gsem).wait()

        def fire_wb(l, b):
            for tr in range(D // 8):
                pltpu.async_copy(
                    tbuf[b].at[pl.ds(tr * 8 * BW, 8 * BW)],
                    out_hbm.at[l, tr, wid], wsem[b])

        def wait_wb(b):
            for tr in range(D // 8):
                pltpu.make_async_copy(
                    tbuf[b].at[pl.ds(tr * 8 * BW, 8 * BW)],
                    out_hbm.at[0, tr, 0], wsem[b]).wait()

        pltpu.sync_copy(idx_hbm.at[pl.ds(base, BW), :], idx_v)

        offs = list(range(0, L - 15, 16)) + [L - 16]

        def idx_t_body(j, carry):
            for off in offs:
                v = idx_v[j, pl.ds(off, 16)]
                plsc.store_scatter(idxT_v, [(lane + off) * BW + j], v + v)
            return carry

        lax.fori_loop(0, BW, idx_t_body, 0)

        for b in range(NBUF):
            fire_gather(b, b)

        def body(i, carry):
            for b in range(NBUF):
                l = i * NBUF + b
                wait_gather(b)

                @pl.when(l >= NBUF)
                def _():
                    wait_wb(b)

                def tr_body(m, carry2):
                    perm = lane + m
                    perm = jnp.where(perm < 16, perm, perm - 16)
                    for jp in range(0, BW, 32):
                        batch = []
                        for j0 in (jp, jp + 16):
                            rowv = perm + j0
                            for k in range(D // 16):
                                v = plsc.load_gather(
                                    gbuf[b], [rowv, lane + 16 * k])
                                batch.append(
                                    ((lane + 16 * k) * BW + rowv, v))
                        for sidx, v in batch:
                            plsc.store_scatter(tbuf[b], [sidx], v)
                    return carry2

                lax.fori_loop(0, 16, tr_body, 0)
                fire_wb(l, b)
                nxt = l + NBUF

                @pl.when(nxt < L)
                def _():
                    fire_gather(nxt, b)

            return carry

        lax.fori_loop(0, L // NBUF, body, 0)
        for b in range(NBUF):
            wait_wb(b)

    return gather_kernel


def kernel(x, table, W, b):
    B, L = x.shape
    V, D = table.shape
    proj = _make_project(V, D, 25600)(table.T, W, b.reshape(1, D))
    out4 = _make_gather(V, D, B, L)(proj.reshape(2 * V, D), x.astype(jnp.int32))
    out5 = out4.reshape(L, D // 8, B // 128, 8, 128)
    return out5.transpose(2, 4, 0, 1, 3).reshape(B, L, D)


# speedup vs baseline: 5.5320x; 1.1335x over previous
import functools

import jax
import jax.numpy as jnp
from jax import lax
from jax.experimental import pallas as pl
from jax.experimental.pallas import tpu as pltpu
from jax.experimental.pallas import tpu_sc as plsc


def _proj_body(tableT_ref, w_ref, b_ref, out_ref):
    out_ref[:, pl.ds(0, w_ref.shape[0])] = lax.dot_general(
        tableT_ref[...], w_ref[...], (((0,), (0,)), ((), ())),
        preferred_element_type=jnp.float32,
    ) + b_ref[...]


@functools.lru_cache(maxsize=None)
def _make_project(V, D, blk):
    grid = (V + blk - 1) // blk
    return pl.pallas_call(
        _proj_body,
        grid=(grid,),
        in_specs=[
            pl.BlockSpec((D, blk), lambda i: (0, i)),
            pl.BlockSpec((D, D), lambda i: (0, 0)),
            pl.BlockSpec((1, D), lambda i: (0, 0)),
        ],
        out_specs=pl.BlockSpec((blk, 2 * D), lambda i: (i, 0)),
        out_shape=jax.ShapeDtypeStruct((V, 2 * D), jnp.float32),
    )


@functools.lru_cache(maxsize=None)
def _make_gather(V, D, B, L):
    info = plsc.get_sparse_core_info()
    NC, NS, LN = info.num_cores, info.num_subcores, info.num_lanes
    NW = NC * NS
    BW = B // NW
    assert BW == 128 and D == 64 and LN == 16
    NBUF = 4
    mesh = plsc.VectorSubcoreMesh(core_axis_name="c", subcore_axis_name="s")

    @functools.partial(
        pl.kernel,
        mesh=mesh,
        compiler_params=pltpu.CompilerParams(
            use_tc_tiling_on_sc=False, needs_layout_passes=False),
        out_type=jax.ShapeDtypeStruct((L, D // 8, NW, 8 * BW), jnp.float32),
        scratch_types=[
            pltpu.VMEM((BW, L), jnp.int32),
            pltpu.VMEM((L * BW,), jnp.int32),
            pltpu.VMEM((BW, D), jnp.float32),
            pltpu.VMEM((BW, D), jnp.float32),
            pltpu.VMEM((BW, D), jnp.float32),
            pltpu.VMEM((BW, D), jnp.float32),
            pltpu.VMEM((D // 8, 8 * BW), jnp.float32),
            pltpu.VMEM((D // 8, 8 * BW), jnp.float32),
            pltpu.VMEM((D // 8, 8 * BW), jnp.float32),
            pltpu.VMEM((D // 8, 8 * BW), jnp.float32),
            pltpu.SemaphoreType.DMA,
            pltpu.SemaphoreType.DMA,
            pltpu.SemaphoreType.DMA,
            pltpu.SemaphoreType.DMA,
            pltpu.SemaphoreType.DMA,
        ],
    )
    def gather_kernel(table_hbm, idx_hbm, out_hbm, idx_v, idxT_v,
                      g0, g1, g2, g3, t0, t1, t2, t3, gsem, w0, w1, w2, w3):
        gbuf = (g0, g1, g2, g3)
        tbuf = (t0, t1, t2, t3)
        wsem = (w0, w1, w2, w3)
        wid = lax.axis_index("s") * NC + lax.axis_index("c")
        base = wid * BW
        lane = lax.iota(jnp.int32, 16)

        def fire_gather(l, b):
            pltpu.async_copy(
                table_hbm.at[idxT_v.at[pl.ds(l * BW, BW)]], gbuf[b], gsem)

        def wait_gather(b):
            pltpu.make_async_copy(
                table_hbm.at[pl.ds(0, BW)], gbuf[b], gsem).wait()

        def fire_wb(l, b):
            pltpu.async_copy(tbuf[b], out_hbm.at[l, :, wid], wsem[b])

        def wait_wb(b):
            pltpu.make_async_copy(
                tbuf[b], out_hbm.at[0, :, 0], wsem[b]).wait()

        pltpu.sync_copy(idx_hbm.at[pl.ds(base, BW), :], idx_v)

        offs = list(range(0, L - 15, 16)) + [L - 16]

        def idx_t_body(j, carry):
            batch = [((lane + off) * BW + j, idx_v[j, pl.ds(off, 16)])
                     for off in offs]
            for sidx, v in batch:
                plsc.store_scatter(idxT_v, [sidx], v + v)
            return carry

        lax.fori_loop(0, BW, idx_t_body, 0)

        for b in range(NBUF):
            fire_gather(b, b)

        def body(i, carry):
            for b in range(NBUF):
                l = i * NBUF + b
                wait_gather(b)

                @pl.when(l >= NBUF)
                def _():
                    wait_wb(b)

                def tr_body(m, carry2):
                    perm = lane + m
                    perm = jnp.where(perm < 16, perm, perm - 16)
                    for jp in range(0, BW, 32):
                        batch = []
                        for j0 in (jp, jp + 16):
                            rowv = perm + j0
                            for k in range(D // 16):
                                d = lane + 16 * k
                                v = plsc.load_gather(gbuf[b], [rowv, d])
                                batch.append(
                                    (d // 8, (d % 8) * BW + rowv, v))
                        for trv, innerv, v in batch:
                            plsc.store_scatter(tbuf[b], [trv, innerv], v)
                    return carry2

                lax.fori_loop(0, 16, tr_body, 0)
                fire_wb(l, b)
                nxt = l + NBUF

                @pl.when(nxt < L)
                def _():
                    fire_gather(nxt, b)

            return carry

        lax.fori_loop(0, L // NBUF, body, 0)
        for b in range(NBUF):
            wait_wb(b)

    return gather_kernel


def kernel(x, table, W, b):
    B, L = x.shape
    V, D = table.shape
    proj = _make_project(V, D, 25600)(table.T, W, b.reshape(1, D))
    out4 = _make_gather(V, D, B, L)(proj.reshape(2 * V, D), x.astype(jnp.int32))
    out5 = out4.reshape(L, D // 8, B // 128, 8, 128)
    return out5.transpose(2, 4, 0, 1, 3).reshape(B, L, D)
